# baseline (speedup 1.0000x reference)
import functools

import jax
import jax.numpy as jnp
from jax import lax
from jax.experimental import pallas as pl
from jax.experimental.pallas import tpu as pltpu
from jax.experimental.pallas import tpu_sc as plsc

N = 10000
D = 128
H3 = 384
H = 256
NC, NS, L = 2, 16, 16
NW = NC * NS
CHUNK = 128
CPT = 80
TE = CHUNK * CPT
E_PAD = NW * TE
N_ACC = 10240
STRIPE = N_ACC // NS
PAD_ROW = N

def _fill(buf, value):
    val = jnp.full((L,), value, jnp.float32)

    @pl.loop(0, CHUNK)
    def _(r):
        @pl.loop(0, D // L)
        def _(j):
            buf[r, pl.ds(j * L, L)] = val


def _deg_body(dst2_hbm, out_hbm, dstall, ones_v, acc):
    c = lax.axis_index("c")
    s = lax.axis_index("s")
    wid = c * NS + s

    pltpu.sync_copy(dst2_hbm.at[pl.ds(wid * CPT, CPT)], dstall)
    _fill(ones_v, 0.0)

    @pl.loop(0, STRIPE // CHUNK)
    def _(z):
        pltpu.sync_copy(ones_v, acc.at[pl.ds(s * STRIPE + z * CHUNK, CHUNK)])

    _fill(ones_v, 1.0)
    plsc.subcore_barrier()

    @pl.loop(0, CPT)
    def _(k):
        pltpu.sync_copy(ones_v, acc.at[dstall.at[k]], add=True)

    plsc.subcore_barrier()
    pltpu.sync_copy(acc.at[pl.ds(s * STRIPE, STRIPE)],
                    out_hbm.at[c].at[pl.ds(s * STRIPE, STRIPE)])


@functools.cache
def _deg_call():
    return pl.kernel(
        _deg_body,
        out_type=jax.ShapeDtypeStruct((NC, N_ACC, D), jnp.float32),
        mesh=plsc.VectorSubcoreMesh(core_axis_name="c", subcore_axis_name="s"),
        scratch_types=[
            pltpu.VMEM((CPT, CHUNK), jnp.int32),
            pltpu.VMEM((CHUNK, D), jnp.float32),
            pltpu.VMEM_SHARED((N_ACC, D), jnp.float32),
        ],
    )


def _prop_body(src2_hbm, dst1_hbm, y_hbm, out_hbm,
               srcall, dstv0, dstv1, rows0, rows1, acc,
               semg0, semg1, semd0, semd1):
    c = lax.axis_index("c")
    s = lax.axis_index("s")
    wid = c * NS + s
    base = wid * TE

    pltpu.sync_copy(src2_hbm.at[pl.ds(wid * CPT, CPT)], srcall)
    _fill(rows0, 0.0)

    @pl.loop(0, STRIPE // CHUNK)
    def _(z):
        pltpu.sync_copy(rows0, acc.at[pl.ds(s * STRIPE + z * CHUNK, CHUNK)])

    plsc.subcore_barrier()

    pltpu.async_copy(dst1_hbm.at[pl.ds(base, CHUNK)], dstv0, semd0)
    pltpu.async_copy(y_hbm.at[srcall.at[0]], rows0, semg0)
    pltpu.async_copy(dst1_hbm.at[pl.ds(base + CHUNK, CHUNK)], dstv1, semd1)

    @pl.loop(0, CPT, step=2)
    def _(k):
        pltpu.async_copy(y_hbm.at[srcall.at[k + 1]], rows1, semg1)

        pltpu.make_async_copy(dst1_hbm.at[pl.ds(base, CHUNK)], dstv0,
                              semd0).wait()
        pltpu.make_async_copy(y_hbm.at[srcall.at[k]], rows0, semg0).wait()
        pltpu.sync_copy(rows0, acc.at[dstv0], add=True)

        @pl.when(k + 2 < CPT)
        def _():
            pltpu.async_copy(dst1_hbm.at[pl.ds(base + (k + 2) * CHUNK, CHUNK)],
                             dstv0, semd0)
            pltpu.async_copy(y_hbm.at[srcall.at[k + 2]], rows0, semg0)

        pltpu.make_async_copy(dst1_hbm.at[pl.ds(base, CHUNK)], dstv1,
                              semd1).wait()
        pltpu.make_async_copy(y_hbm.at[srcall.at[k + 1]], rows1, semg1).wait()
        pltpu.sync_copy(rows1, acc.at[dstv1], add=True)

        @pl.when(k + 3 < CPT)
        def _():
            pltpu.async_copy(dst1_hbm.at[pl.ds(base + (k + 3) * CHUNK, CHUNK)],
                             dstv1, semd1)

    plsc.subcore_barrier()
    pltpu.sync_copy(acc.at[pl.ds(s * STRIPE, STRIPE)],
                    out_hbm.at[c].at[pl.ds(s * STRIPE, STRIPE)])


@functools.cache
def _prop_call():
    return pl.kernel(
        _prop_body,
        out_type=jax.ShapeDtypeStruct((NC, N_ACC, D), jnp.float32),
        mesh=plsc.VectorSubcoreMesh(core_axis_name="c", subcore_axis_name="s"),
        scratch_types=[
            pltpu.VMEM((CPT, CHUNK), jnp.int32),
            pltpu.VMEM((CHUNK,), jnp.int32),
            pltpu.VMEM((CHUNK,), jnp.int32),
            pltpu.VMEM((CHUNK, D), jnp.float32),
            pltpu.VMEM((CHUNK, D), jnp.float32),
            pltpu.VMEM_SHARED((N_ACC, D), jnp.float32),
            pltpu.SemaphoreType.DMA,
            pltpu.SemaphoreType.DMA,
            pltpu.SemaphoreType.DMA,
            pltpu.SemaphoreType.DMA,
        ],
    )


_BM = 1000
_NB = N // _BM
_NF = float(N)


def _prep_body(dp0, dp1, x_ref, y1_ref, dinv_ref):
    deg = dp0[0][:, 0:1] + dp1[0][:, 0:1]
    dinv = jnp.where(deg > 0.0, lax.rsqrt(jnp.where(deg > 0.0, deg, 1.0)), 0.0)
    dinv_ref[...] = dinv
    y1_ref[...] = x_ref[...] * dinv


def _mid_body(zp0, zp1, dinv_ref, h1_ref, y2_ref):
    dinv = dinv_ref[...]
    h1 = (zp0[0] + zp1[0]) * dinv
    h1_ref[...] = h1
    y2_ref[...] = h1 * dinv


def _d1_body(zp0, zp1, dinv_ref, x_ref, h1_ref,
             w0, b0, w1, b1, w2, b2, out_ref, st_ref):
    i = pl.program_id(0)
    dot = functools.partial(jnp.dot, preferred_element_type=jnp.float32,
                            precision=lax.Precision.HIGHEST)
    h2 = (zp0[0] + zp1[0]) * dinv_ref[...]
    o = jnp.concatenate([
        dot(x_ref[...], w0[...]) + b0[...],
        dot(h1_ref[...], w1[...]) + b1[...],
        dot(h2, w2[...]) + b2[...],
    ], axis=1)
    out_ref[...] = o

    @pl.when(i == 0)
    def _():
        st_ref[...] = jnp.zeros_like(st_ref)

    st_ref[0:1, :] += jnp.sum(o, axis=0, keepdims=True)
    st_ref[1:2, :] += jnp.sum(o * o, axis=0, keepdims=True)


def _d2_body(out_ref, st_ref, g1, bb1, l1w, l1b, l2w, l2b, h_ref, st2_ref):
    i = pl.program_id(0)
    dot = functools.partial(jnp.dot, preferred_element_type=jnp.float32,
                            precision=lax.Precision.HIGHEST)
    mu = st_ref[0:1, :] / _NF
    var = st_ref[1:2, :] / _NF - mu * mu
    inv = lax.rsqrt(var + 1e-5)
    hb = (out_ref[...] - mu) * inv * g1[...] + bb1[...]
    t = jnp.maximum(dot(hb, l1w[...]) + l1b[...], 0.0)
    y = dot(t, l2w[...]) + l2b[...]
    h = hb[:, :D] + y
    h_ref[...] = h

    @pl.when(i == 0)
    def _():
        st2_ref[...] = jnp.zeros_like(st2_ref)

    st2_ref[0:1, :] += jnp.sum(h, axis=0, keepdims=True)
    st2_ref[1:2, :] += jnp.sum(h * h, axis=0, keepdims=True)


def _d3_body(h_ref, st2_ref, g2, bb2, o_ref):
    mu = st2_ref[0:1, :] / _NF
    var = st2_ref[1:2, :] / _NF - mu * mu
    inv = lax.rsqrt(var + 1e-5)
    o_ref[...] = (h_ref[...] - mu) * inv * g2[...] + bb2[...]


def _row_spec(bn, bd):
    return pl.BlockSpec((bn, bd), lambda i: (i, 0))


def _part_spec(part, bn, bd):
    return pl.BlockSpec((1, bn, bd), lambda i, p=part: (p, i, 0))


def _full_spec(shape):
    return pl.BlockSpec(shape, lambda i: tuple(0 for _ in shape))


@jax.jit
def kernel(x, edge_index, W0, b0, W1, b1, W2, b2, bn1_g, bn1_b,
           l1_W, l1_b, l2_W, l2_b, bn2_g, bn2_b):
    E = edge_index.shape[1]
    pad = jnp.full((E_PAD - E,), PAD_ROW, jnp.int32)
    src = jnp.concatenate([edge_index[0].astype(jnp.int32), pad])
    dst1 = jnp.concatenate([edge_index[1].astype(jnp.int32), pad])
    src2 = src.reshape(E_PAD // CHUNK, CHUNK)
    dst2 = dst1.reshape(E_PAD // CHUNK, CHUNK)


    b0r = b0.reshape(1, D)
    b1r = b1.reshape(1, D)
    b2r = b2.reshape(1, D)
    g1r = bn1_g.reshape(1, H3)
    bb1r = bn1_b.reshape(1, H3)
    l1br = l1_b.reshape(1, H)
    l2br = l2_b.reshape(1, D)
    g2r = bn2_g.reshape(1, D)
    bb2r = bn2_b.reshape(1, D)

    degp = _deg_call()(dst2)

    y1, dinv = pl.pallas_call(
        _prep_body,
        grid=(_NB,),
        in_specs=[_part_spec(0, _BM, D), _part_spec(1, _BM, D),
                  _row_spec(_BM, D)],
        out_specs=[_row_spec(_BM, D), _row_spec(_BM, 1)],
        out_shape=[jax.ShapeDtypeStruct((N_ACC, D), jnp.float32),
                   jax.ShapeDtypeStruct((N, 1), jnp.float32)],
    )(degp, degp, x)

    z1p = _prop_call()(src2, dst1, y1)

    h1, y2 = pl.pallas_call(
        _mid_body,
        grid=(_NB,),
        in_specs=[_part_spec(0, _BM, D), _part_spec(1, _BM, D),
                  _row_spec(_BM, 1)],
        out_specs=[_row_spec(_BM, D), _row_spec(_BM, D)],
        out_shape=[jax.ShapeDtypeStruct((N, D), jnp.float32),
                   jax.ShapeDtypeStruct((N_ACC, D), jnp.float32)],
    )(z1p, z1p, dinv)

    z2p = _prop_call()(src2, dst1, y2)

    out, st1 = pl.pallas_call(
        _d1_body,
        grid=(_NB,),
        in_specs=[_part_spec(0, _BM, D), _part_spec(1, _BM, D),
                  _row_spec(_BM, 1), _row_spec(_BM, D), _row_spec(_BM, D),
                  _full_spec((D, D)), _full_spec((1, D)),
                  _full_spec((D, D)), _full_spec((1, D)),
                  _full_spec((D, D)), _full_spec((1, D))],
        out_specs=[_row_spec(_BM, H3), _full_spec((8, H3))],
        out_shape=[jax.ShapeDtypeStruct((N, H3), jnp.float32),
                   jax.ShapeDtypeStruct((8, H3), jnp.float32)],
    )(z2p, z2p, dinv, x, h1, W0, b0r, W1, b1r, W2, b2r)

    h, st2 = pl.pallas_call(
        _d2_body,
        grid=(_NB,),
        in_specs=[_row_spec(_BM, H3), _full_spec((8, H3)),
                  _full_spec((1, H3)), _full_spec((1, H3)),
                  _full_spec((H3, H)), _full_spec((1, H)),
                  _full_spec((H, D)), _full_spec((1, D))],
        out_specs=[_row_spec(_BM, D), _full_spec((8, D))],
        out_shape=[jax.ShapeDtypeStruct((N, D), jnp.float32),
                   jax.ShapeDtypeStruct((8, D), jnp.float32)],
    )(out, st1, g1r, bb1r, l1_W, l1br, l2_W, l2br)

    final = pl.pallas_call(
        _d3_body,
        grid=(_NB,),
        in_specs=[_row_spec(_BM, D), _full_spec((8, D)),
                  _full_spec((1, D)), _full_spec((1, D))],
        out_specs=_row_spec(_BM, D),
        out_shape=jax.ShapeDtypeStruct((N, D), jnp.float32),
    )(h, st2, g2r, bb2r)

    return final


# speedup vs baseline: 2.8431x; 2.8431x over previous
import functools

import jax
import jax.numpy as jnp
from jax import lax
from jax.experimental import pallas as pl
from jax.experimental.pallas import tpu as pltpu
from jax.experimental.pallas import tpu_sc as plsc

N = 10000
D = 128
H3 = 384
H = 256
NC, NS, L = 2, 16, 16
NW = NC * NS
CHUNK = 128
CPT = 80
TE = CHUNK * CPT
E_PAD = NW * TE
N_ACC = 10240
STRIPE = N_ACC // NS
PAD_ROW = N

def _fill(buf, value):
    val = jnp.full((L,), value, jnp.float32)

    @pl.loop(0, CHUNK)
    def _(r):
        @pl.loop(0, D // L)
        def _(j):
            buf[r, pl.ds(j * L, L)] = val


def _deg_body(dst2_hbm, out_hbm, dstall, ones_v, acc):
    c = lax.axis_index("c")
    s = lax.axis_index("s")
    wid = c * NS + s

    pltpu.sync_copy(dst2_hbm.at[pl.ds(wid * CPT, CPT)], dstall)
    _fill(ones_v, 0.0)

    @pl.loop(0, STRIPE // CHUNK)
    def _(z):
        pltpu.sync_copy(ones_v, acc.at[pl.ds(s * STRIPE + z * CHUNK, CHUNK)])

    _fill(ones_v, 1.0)
    plsc.subcore_barrier()

    @pl.loop(0, CPT)
    def _(k):
        pltpu.sync_copy(ones_v, acc.at[dstall.at[k]], add=True)

    plsc.subcore_barrier()
    pltpu.sync_copy(acc.at[pl.ds(s * STRIPE, STRIPE)],
                    out_hbm.at[c].at[pl.ds(s * STRIPE, STRIPE)])


@functools.cache
def _deg_call():
    return pl.kernel(
        _deg_body,
        out_type=jax.ShapeDtypeStruct((NC, N_ACC, D), jnp.float32),
        mesh=plsc.VectorSubcoreMesh(core_axis_name="c", subcore_axis_name="s"),
        scratch_types=[
            pltpu.VMEM((CPT, CHUNK), jnp.int32),
            pltpu.VMEM((CHUNK, D), jnp.float32),
            pltpu.VMEM_SHARED((N_ACC, D), jnp.float32),
        ],
    )


def _prop_body(src2_hbm, dst1_hbm, y_hbm, out_hbm,
               srcall, dstv0, dstv1, rows0, rows1, acc,
               semg0, semg1, semd0, semd1):
    c = lax.axis_index("c")
    s = lax.axis_index("s")
    wid = c * NS + s
    base = wid * TE

    pltpu.sync_copy(src2_hbm.at[pl.ds(wid * CPT, CPT)], srcall)
    _fill(rows0, 0.0)

    @pl.loop(0, STRIPE // CHUNK)
    def _(z):
        pltpu.sync_copy(rows0, acc.at[pl.ds(s * STRIPE + z * CHUNK, CHUNK)])

    plsc.subcore_barrier()

    pltpu.async_copy(dst1_hbm.at[pl.ds(base, CHUNK)], dstv0, semd0)
    pltpu.async_copy(y_hbm.at[srcall.at[0]], rows0, semg0)
    pltpu.async_copy(dst1_hbm.at[pl.ds(base + CHUNK, CHUNK)], dstv1, semd1)

    @pl.loop(0, CPT, step=2)
    def _(k):
        pltpu.async_copy(y_hbm.at[srcall.at[k + 1]], rows1, semg1)

        pltpu.make_async_copy(dst1_hbm.at[pl.ds(base, CHUNK)], dstv0,
                              semd0).wait()
        pltpu.make_async_copy(y_hbm.at[srcall.at[k]], rows0, semg0).wait()
        pltpu.sync_copy(rows0, acc.at[dstv0], add=True)

        @pl.when(k + 2 < CPT)
        def _():
            pltpu.async_copy(dst1_hbm.at[pl.ds(base + (k + 2) * CHUNK, CHUNK)],
                             dstv0, semd0)
            pltpu.async_copy(y_hbm.at[srcall.at[k + 2]], rows0, semg0)

        pltpu.make_async_copy(dst1_hbm.at[pl.ds(base, CHUNK)], dstv1,
                              semd1).wait()
        pltpu.make_async_copy(y_hbm.at[srcall.at[k + 1]], rows1, semg1).wait()
        pltpu.sync_copy(rows1, acc.at[dstv1], add=True)

        @pl.when(k + 3 < CPT)
        def _():
            pltpu.async_copy(dst1_hbm.at[pl.ds(base + (k + 3) * CHUNK, CHUNK)],
                             dstv1, semd1)

    plsc.subcore_barrier()
    pltpu.sync_copy(acc.at[pl.ds(s * STRIPE, STRIPE)],
                    out_hbm.at[c].at[pl.ds(s * STRIPE, STRIPE)])


@functools.cache
def _prop_call():
    return pl.kernel(
        _prop_body,
        out_type=jax.ShapeDtypeStruct((NC, N_ACC, D), jnp.float32),
        mesh=plsc.VectorSubcoreMesh(core_axis_name="c", subcore_axis_name="s"),
        scratch_types=[
            pltpu.VMEM((CPT, CHUNK), jnp.int32),
            pltpu.VMEM((CHUNK,), jnp.int32),
            pltpu.VMEM((CHUNK,), jnp.int32),
            pltpu.VMEM((CHUNK, D), jnp.float32),
            pltpu.VMEM((CHUNK, D), jnp.float32),
            pltpu.VMEM_SHARED((N_ACC, D), jnp.float32),
            pltpu.SemaphoreType.DMA,
            pltpu.SemaphoreType.DMA,
            pltpu.SemaphoreType.DMA,
            pltpu.SemaphoreType.DMA,
        ],
    )


_BM = 1000
_NB = N // _BM
_NF = float(N)


def _prep_body(dp0, dp1, x_ref, y1_ref, dinv_ref):
    deg = dp0[0][:, 0:1] + dp1[0][:, 0:1]
    dinv = jnp.where(deg > 0.0, lax.rsqrt(jnp.where(deg > 0.0, deg, 1.0)), 0.0)
    dinv_ref[...] = dinv
    y1_ref[...] = x_ref[...] * dinv


def _mid_body(zp0, zp1, dinv_ref, h1_ref, y2_ref):
    dinv = dinv_ref[...]
    h1 = (zp0[0] + zp1[0]) * dinv
    h1_ref[...] = h1
    y2_ref[...] = h1 * dinv


def _d1_body(zp0, zp1, dinv_ref, x_ref, h1_ref,
             w0, b0, w1, b1, w2, b2, out_ref, st_ref):
    i = pl.program_id(0)
    dot = functools.partial(jnp.dot, preferred_element_type=jnp.float32,
                            precision=lax.Precision.HIGHEST)
    h2 = (zp0[0] + zp1[0]) * dinv_ref[...]
    o = jnp.concatenate([
        dot(x_ref[...], w0[...]) + b0[...],
        dot(h1_ref[...], w1[...]) + b1[...],
        dot(h2, w2[...]) + b2[...],
    ], axis=1)
    out_ref[...] = o

    @pl.when(i == 0)
    def _():
        st_ref[...] = jnp.zeros_like(st_ref)

    st_ref[0:1, :] += jnp.sum(o, axis=0, keepdims=True)
    st_ref[1:2, :] += jnp.sum(o * o, axis=0, keepdims=True)


def _d2_body(out_ref, st_ref, g1, bb1, l1w, l1b, l2w, l2b, h_ref, st2_ref):
    i = pl.program_id(0)
    dot = functools.partial(jnp.dot, preferred_element_type=jnp.float32,
                            precision=lax.Precision.HIGHEST)
    mu = st_ref[0:1, :] / _NF
    var = st_ref[1:2, :] / _NF - mu * mu
    inv = lax.rsqrt(var + 1e-5)
    hb = (out_ref[...] - mu) * inv * g1[...] + bb1[...]
    t = jnp.maximum(dot(hb, l1w[...]) + l1b[...], 0.0)
    y = dot(t, l2w[...]) + l2b[...]
    h = hb[:, :D] + y
    h_ref[...] = h

    @pl.when(i == 0)
    def _():
        st2_ref[...] = jnp.zeros_like(st2_ref)

    st2_ref[0:1, :] += jnp.sum(h, axis=0, keepdims=True)
    st2_ref[1:2, :] += jnp.sum(h * h, axis=0, keepdims=True)


def _d3_body(h_ref, st2_ref, g2, bb2, o_ref):
    mu = st2_ref[0:1, :] / _NF
    var = st2_ref[1:2, :] / _NF - mu * mu
    inv = lax.rsqrt(var + 1e-5)
    o_ref[...] = (h_ref[...] - mu) * inv * g2[...] + bb2[...]


def _row_spec(bn, bd):
    return pl.BlockSpec((bn, bd), lambda i: (i, 0))


def _part_spec(part, bn, bd):
    return pl.BlockSpec((1, bn, bd), lambda i, p=part: (p, i, 0))


def _full_spec(shape):
    return pl.BlockSpec(shape, lambda i: tuple(0 for _ in shape))


@jax.jit
def kernel(x, edge_index, W0, b0, W1, b1, W2, b2, bn1_g, bn1_b,
           l1_W, l1_b, l2_W, l2_b, bn2_g, bn2_b):
    E = edge_index.shape[1]
    pad_src = jnp.arange(E_PAD - E, dtype=jnp.int32) % N
    pad_dst = jnp.full((E_PAD - E,), PAD_ROW, jnp.int32)
    src = jnp.concatenate([edge_index[0].astype(jnp.int32), pad_src])
    dst1 = jnp.concatenate([edge_index[1].astype(jnp.int32), pad_dst])
    src2 = src.reshape(E_PAD // CHUNK, CHUNK)
    dst2 = dst1.reshape(E_PAD // CHUNK, CHUNK)


    b0r = b0.reshape(1, D)
    b1r = b1.reshape(1, D)
    b2r = b2.reshape(1, D)
    g1r = bn1_g.reshape(1, H3)
    bb1r = bn1_b.reshape(1, H3)
    l1br = l1_b.reshape(1, H)
    l2br = l2_b.reshape(1, D)
    g2r = bn2_g.reshape(1, D)
    bb2r = bn2_b.reshape(1, D)

    degp = _deg_call()(dst2)

    y1, dinv = pl.pallas_call(
        _prep_body,
        grid=(_NB,),
        in_specs=[_part_spec(0, _BM, D), _part_spec(1, _BM, D),
                  _row_spec(_BM, D)],
        out_specs=[_row_spec(_BM, D), _row_spec(_BM, 1)],
        out_shape=[jax.ShapeDtypeStruct((N_ACC, D), jnp.float32),
                   jax.ShapeDtypeStruct((N, 1), jnp.float32)],
    )(degp, degp, x)

    z1p = _prop_call()(src2, dst1, y1)

    h1, y2 = pl.pallas_call(
        _mid_body,
        grid=(_NB,),
        in_specs=[_part_spec(0, _BM, D), _part_spec(1, _BM, D),
                  _row_spec(_BM, 1)],
        out_specs=[_row_spec(_BM, D), _row_spec(_BM, D)],
        out_shape=[jax.ShapeDtypeStruct((N, D), jnp.float32),
                   jax.ShapeDtypeStruct((N_ACC, D), jnp.float32)],
    )(z1p, z1p, dinv)

    z2p = _prop_call()(src2, dst1, y2)

    out, st1 = pl.pallas_call(
        _d1_body,
        grid=(_NB,),
        in_specs=[_part_spec(0, _BM, D), _part_spec(1, _BM, D),
                  _row_spec(_BM, 1), _row_spec(_BM, D), _row_spec(_BM, D),
                  _full_spec((D, D)), _full_spec((1, D)),
                  _full_spec((D, D)), _full_spec((1, D)),
                  _full_spec((D, D)), _full_spec((1, D))],
        out_specs=[_row_spec(_BM, H3), _full_spec((8, H3))],
        out_shape=[jax.ShapeDtypeStruct((N, H3), jnp.float32),
                   jax.ShapeDtypeStruct((8, H3), jnp.float32)],
    )(z2p, z2p, dinv, x, h1, W0, b0r, W1, b1r, W2, b2r)

    h, st2 = pl.pallas_call(
        _d2_body,
        grid=(_NB,),
        in_specs=[_row_spec(_BM, H3), _full_spec((8, H3)),
                  _full_spec((1, H3)), _full_spec((1, H3)),
                  _full_spec((H3, H)), _full_spec((1, H)),
                  _full_spec((H, D)), _full_spec((1, D))],
        out_specs=[_row_spec(_BM, D), _full_spec((8, D))],
        out_shape=[jax.ShapeDtypeStruct((N, D), jnp.float32),
                   jax.ShapeDtypeStruct((8, D), jnp.float32)],
    )(out, st1, g1r, bb1r, l1_W, l1br, l2_W, l2br)

    final = pl.pallas_call(
        _d3_body,
        grid=(_NB,),
        in_specs=[_row_spec(_BM, D), _full_spec((8, D)),
                  _full_spec((1, D)), _full_spec((1, D))],
        out_specs=_row_spec(_BM, D),
        out_shape=jax.ShapeDtypeStruct((N, D), jnp.float32),
    )(h, st2, g2r, bb2r)

    return final


# speedup vs baseline: 2.9098x; 1.0235x over previous
import functools

import jax
import jax.numpy as jnp
from jax import lax
from jax.experimental import pallas as pl
from jax.experimental.pallas import tpu as pltpu
from jax.experimental.pallas import tpu_sc as plsc

N = 10000
D = 128
H3 = 384
H = 256
NC, NS, L = 2, 16, 16
NW = NC * NS
CHUNK = 128
CPT = 80
TE = CHUNK * CPT
E_PAD = NW * TE
N_ACC = 10240
STRIPE = N_ACC // NS
PAD_ROW = N

def _fill(buf, value):
    val = jnp.full((L,), value, jnp.float32)

    @pl.loop(0, CHUNK)
    def _(r):
        @pl.loop(0, D // L)
        def _(j):
            buf[r, pl.ds(j * L, L)] = val


DW = 128


def _fill_w(buf, value, width):
    val = jnp.full((L,), value, jnp.float32)

    @pl.loop(0, CHUNK)
    def _(r):
        @pl.loop(0, width // L)
        def _(j):
            buf[r, pl.ds(j * L, L)] = val


def _deg_body(dst2_hbm, out_hbm, dstall, ones_v, acc):
    c = lax.axis_index("c")
    s = lax.axis_index("s")
    wid = c * NS + s

    pltpu.sync_copy(dst2_hbm.at[pl.ds(wid * CPT, CPT)], dstall)
    _fill_w(ones_v, 0.0, DW)

    @pl.loop(0, STRIPE // CHUNK)
    def _(z):
        pltpu.sync_copy(ones_v, acc.at[pl.ds(s * STRIPE + z * CHUNK, CHUNK)])

    _fill_w(ones_v, 1.0, DW)
    plsc.subcore_barrier()

    @pl.loop(0, CPT)
    def _(k):
        pltpu.sync_copy(ones_v, acc.at[dstall.at[k]], add=True)

    plsc.subcore_barrier()
    pltpu.sync_copy(acc.at[pl.ds(s * STRIPE, STRIPE)],
                    out_hbm.at[c].at[pl.ds(s * STRIPE, STRIPE)])


@functools.cache
def _deg_call():
    return pl.kernel(
        _deg_body,
        out_type=jax.ShapeDtypeStruct((NC, N_ACC, DW), jnp.float32),
        mesh=plsc.VectorSubcoreMesh(core_axis_name="c", subcore_axis_name="s"),
        scratch_types=[
            pltpu.VMEM((CPT, CHUNK), jnp.int32),
            pltpu.VMEM((CHUNK, DW), jnp.float32),
            pltpu.VMEM_SHARED((N_ACC, DW), jnp.float32),
        ],
    )


def _prop_body(src2_hbm, dst1_hbm, y_hbm, out_hbm,
               srcall, dstv0, dstv1, rows0, rows1, acc,
               semg0, semg1, semd0, semd1):
    c = lax.axis_index("c")
    s = lax.axis_index("s")
    wid = c * NS + s
    base = wid * TE

    pltpu.sync_copy(src2_hbm.at[pl.ds(wid * CPT, CPT)], srcall)
    _fill(rows0, 0.0)

    @pl.loop(0, STRIPE // CHUNK)
    def _(z):
        pltpu.sync_copy(rows0, acc.at[pl.ds(s * STRIPE + z * CHUNK, CHUNK)])

    plsc.subcore_barrier()

    pltpu.async_copy(dst1_hbm.at[pl.ds(base, CHUNK)], dstv0, semd0)
    pltpu.async_copy(y_hbm.at[srcall.at[0]], rows0, semg0)
    pltpu.async_copy(dst1_hbm.at[pl.ds(base + CHUNK, CHUNK)], dstv1, semd1)

    @pl.loop(0, CPT, step=2)
    def _(k):
        pltpu.async_copy(y_hbm.at[srcall.at[k + 1]], rows1, semg1)

        pltpu.make_async_copy(dst1_hbm.at[pl.ds(base, CHUNK)], dstv0,
                              semd0).wait()
        pltpu.make_async_copy(y_hbm.at[srcall.at[k]], rows0, semg0).wait()
        pltpu.sync_copy(rows0, acc.at[dstv0], add=True)

        @pl.when(k + 2 < CPT)
        def _():
            pltpu.async_copy(dst1_hbm.at[pl.ds(base + (k + 2) * CHUNK, CHUNK)],
                             dstv0, semd0)
            pltpu.async_copy(y_hbm.at[srcall.at[k + 2]], rows0, semg0)

        pltpu.make_async_copy(dst1_hbm.at[pl.ds(base, CHUNK)], dstv1,
                              semd1).wait()
        pltpu.make_async_copy(y_hbm.at[srcall.at[k + 1]], rows1, semg1).wait()
        pltpu.sync_copy(rows1, acc.at[dstv1], add=True)

        @pl.when(k + 3 < CPT)
        def _():
            pltpu.async_copy(dst1_hbm.at[pl.ds(base + (k + 3) * CHUNK, CHUNK)],
                             dstv1, semd1)

    plsc.subcore_barrier()
    pltpu.sync_copy(acc.at[pl.ds(s * STRIPE, STRIPE)],
                    out_hbm.at[c].at[pl.ds(s * STRIPE, STRIPE)])


@functools.cache
def _prop_call():
    return pl.kernel(
        _prop_body,
        out_type=jax.ShapeDtypeStruct((NC, N_ACC, D), jnp.float32),
        mesh=plsc.VectorSubcoreMesh(core_axis_name="c", subcore_axis_name="s"),
        scratch_types=[
            pltpu.VMEM((CPT, CHUNK), jnp.int32),
            pltpu.VMEM((CHUNK,), jnp.int32),
            pltpu.VMEM((CHUNK,), jnp.int32),
            pltpu.VMEM((CHUNK, D), jnp.float32),
            pltpu.VMEM((CHUNK, D), jnp.float32),
            pltpu.VMEM_SHARED((N_ACC, D), jnp.float32),
            pltpu.SemaphoreType.DMA,
            pltpu.SemaphoreType.DMA,
            pltpu.SemaphoreType.DMA,
            pltpu.SemaphoreType.DMA,
        ],
    )


_BM = 1000
_NB = N // _BM
_NF = float(N)


def _prep_body(dp0, dp1, x_ref, y1_ref, dinv_ref):
    deg = dp0[0][:, 0:1] + dp1[0][:, 0:1]
    dinv = jnp.where(deg > 0.0, lax.rsqrt(jnp.where(deg > 0.0, deg, 1.0)), 0.0)
    dinv_ref[...] = dinv
    y1_ref[...] = x_ref[...] * dinv


def _mid_body(zp0, zp1, dinv_ref, h1_ref, y2_ref):
    dinv = dinv_ref[...]
    h1 = (zp0[0] + zp1[0]) * dinv
    h1_ref[...] = h1
    y2_ref[...] = h1 * dinv


def _dense_body(zp0, zp1, dinv_ref, x_ref, h1_ref,
                w0, b0, w1, b1, w2, b2, g1, bb1, l1w, l1b, l2w, l2b,
                g2, bb2, o_ref, out_scr, h_scr, st1, st2):
    p = pl.program_id(0)
    i = pl.program_id(1)
    dot = functools.partial(jnp.dot, preferred_element_type=jnp.float32,
                            precision=lax.Precision.HIGHEST)
    rows = pl.ds(i * _BM, _BM)

    @pl.when(p == 0)
    def _():
        @pl.when(i == 0)
        def _():
            st1[...] = jnp.zeros_like(st1)

        h2 = (zp0[0] + zp1[0]) * dinv_ref[...]
        o = jnp.concatenate([
            dot(x_ref[...], w0[...]) + b0[...],
            dot(h1_ref[...], w1[...]) + b1[...],
            dot(h2, w2[...]) + b2[...],
        ], axis=1)
        out_scr[rows, :] = o
        st1[0:1, :] += jnp.sum(o, axis=0, keepdims=True)
        st1[1:2, :] += jnp.sum(o * o, axis=0, keepdims=True)

    @pl.when(p == 1)
    def _():
        @pl.when(i == 0)
        def _():
            st2[...] = jnp.zeros_like(st2)

        mu = st1[0:1, :] / _NF
        var = st1[1:2, :] / _NF - mu * mu
        inv = lax.rsqrt(var + 1e-5)
        hb = (out_scr[rows, :] - mu) * inv * g1[...] + bb1[...]
        t = jnp.maximum(dot(hb, l1w[...]) + l1b[...], 0.0)
        y = dot(t, l2w[...]) + l2b[...]
        h = hb[:, :D] + y
        h_scr[rows, :] = h
        st2[0:1, :] += jnp.sum(h, axis=0, keepdims=True)
        st2[1:2, :] += jnp.sum(h * h, axis=0, keepdims=True)

    @pl.when(p == 2)
    def _():
        mu = st2[0:1, :] / _NF
        var = st2[1:2, :] / _NF - mu * mu
        inv = lax.rsqrt(var + 1e-5)
        o_ref[...] = (h_scr[rows, :] - mu) * inv * g2[...] + bb2[...]


def _row_spec(bn, bd):
    return pl.BlockSpec((bn, bd), lambda i: (i, 0))


def _part_spec(part, bn, bd):
    return pl.BlockSpec((1, bn, bd), lambda i, q=part: (q, i, 0))


def _full_spec(shape):
    return pl.BlockSpec(shape, lambda i: tuple(0 for _ in shape))


def _drow_spec(bn, bd):
    return pl.BlockSpec((bn, bd), lambda p, i: (jnp.where(p == 0, i, 0), 0))


def _dpart_spec(part, bn, bd):
    return pl.BlockSpec((1, bn, bd),
                        lambda p, i, q=part: (q, jnp.where(p == 0, i, 0), 0))


def _dfull_spec(shape):
    return pl.BlockSpec(shape, lambda p, i: tuple(0 for _ in shape))


def _dout_spec(bn, bd):
    return pl.BlockSpec((bn, bd), lambda p, i: (jnp.where(p == 2, i, 0), 0))


@jax.jit
def kernel(x, edge_index, W0, b0, W1, b1, W2, b2, bn1_g, bn1_b,
           l1_W, l1_b, l2_W, l2_b, bn2_g, bn2_b):
    E = edge_index.shape[1]
    pad_src = jnp.arange(E_PAD - E, dtype=jnp.int32) % N
    pad_dst = jnp.full((E_PAD - E,), PAD_ROW, jnp.int32)
    src = jnp.concatenate([edge_index[0].astype(jnp.int32), pad_src])
    dst1 = jnp.concatenate([edge_index[1].astype(jnp.int32), pad_dst])
    src2 = src.reshape(E_PAD // CHUNK, CHUNK)
    dst2 = dst1.reshape(E_PAD // CHUNK, CHUNK)


    b0r = b0.reshape(1, D)
    b1r = b1.reshape(1, D)
    b2r = b2.reshape(1, D)
    g1r = bn1_g.reshape(1, H3)
    bb1r = bn1_b.reshape(1, H3)
    l1br = l1_b.reshape(1, H)
    l2br = l2_b.reshape(1, D)
    g2r = bn2_g.reshape(1, D)
    bb2r = bn2_b.reshape(1, D)

    degp = _deg_call()(dst2)

    y1, dinv = pl.pallas_call(
        _prep_body,
        grid=(_NB,),
        in_specs=[_part_spec(0, _BM, DW), _part_spec(1, _BM, DW),
                  _row_spec(_BM, D)],
        out_specs=[_row_spec(_BM, D), _row_spec(_BM, 1)],
        out_shape=[jax.ShapeDtypeStruct((N_ACC, D), jnp.float32),
                   jax.ShapeDtypeStruct((N, 1), jnp.float32)],
    )(degp, degp, x)

    z1p = _prop_call()(src2, dst1, y1)

    h1, y2 = pl.pallas_call(
        _mid_body,
        grid=(_NB,),
        in_specs=[_part_spec(0, _BM, D), _part_spec(1, _BM, D),
                  _row_spec(_BM, 1)],
        out_specs=[_row_spec(_BM, D), _row_spec(_BM, D)],
        out_shape=[jax.ShapeDtypeStruct((N, D), jnp.float32),
                   jax.ShapeDtypeStruct((N_ACC, D), jnp.float32)],
    )(z1p, z1p, dinv)

    z2p = _prop_call()(src2, dst1, y2)

    final = pl.pallas_call(
        _dense_body,
        grid=(3, _NB),
        in_specs=[_dpart_spec(0, _BM, D), _dpart_spec(1, _BM, D),
                  _drow_spec(_BM, 1), _drow_spec(_BM, D), _drow_spec(_BM, D),
                  _dfull_spec((D, D)), _dfull_spec((1, D)),
                  _dfull_spec((D, D)), _dfull_spec((1, D)),
                  _dfull_spec((D, D)), _dfull_spec((1, D)),
                  _dfull_spec((1, H3)), _dfull_spec((1, H3)),
                  _dfull_spec((H3, H)), _dfull_spec((1, H)),
                  _dfull_spec((H, D)), _dfull_spec((1, D)),
                  _dfull_spec((1, D)), _dfull_spec((1, D))],
        out_specs=_dout_spec(_BM, D),
        out_shape=jax.ShapeDtypeStruct((N, D), jnp.float32),
        scratch_shapes=[pltpu.VMEM((N, H3), jnp.float32),
                        pltpu.VMEM((N, D), jnp.float32),
                        pltpu.VMEM((8, H3), jnp.float32),
                        pltpu.VMEM((8, D), jnp.float32)],
    )(z2p, z2p, dinv, x, h1, W0, b0r, W1, b1r, W2, b2r,
      g1r, bb1r, l1_W, l1br, l2_W, l2br, g2r, bb2r)

    return final


# speedup vs baseline: 3.2801x; 1.1272x over previous
import functools

import jax
import jax.numpy as jnp
from jax import lax
from jax.experimental import pallas as pl
from jax.experimental.pallas import tpu as pltpu
from jax.experimental.pallas import tpu_sc as plsc

N = 10000
D = 128
H3 = 384
H = 256
NC, NS, L = 2, 16, 16
NW = NC * NS
CHUNK = 128
CPT = 80
TE = CHUNK * CPT
E_PAD = NW * TE
N_ACC = 10240
STRIPE = N_ACC // NS
PAD_ROW = N

def _fill(buf, value):
    val = jnp.full((L,), value, jnp.float32)

    @pl.loop(0, CHUNK)
    def _(r):
        @pl.loop(0, D // L)
        def _(j):
            buf[r, pl.ds(j * L, L)] = val


DW = 128
DEG_OUT = 128


def _fill_w(buf, value, width):
    val = jnp.full((L,), value, jnp.float32)

    @pl.loop(0, CHUNK)
    def _(r):
        @pl.loop(0, width // L)
        def _(j):
            buf[r, pl.ds(j * L, L)] = val


def _deg_body(dst2_hbm, out_hbm, dstall, ones_v, acc):
    c = lax.axis_index("c")
    s = lax.axis_index("s")
    wid = c * NS + s

    pltpu.sync_copy(dst2_hbm.at[pl.ds(wid * CPT, CPT)], dstall)
    _fill_w(ones_v, 0.0, DW)

    @pl.loop(0, STRIPE // CHUNK)
    def _(z):
        pltpu.sync_copy(ones_v, acc.at[pl.ds(s * STRIPE + z * CHUNK, CHUNK)])

    _fill_w(ones_v, 1.0, DW)
    plsc.subcore_barrier()

    @pl.loop(0, CPT)
    def _(k):
        pltpu.sync_copy(ones_v, acc.at[dstall.at[k]], add=True)

    plsc.subcore_barrier()
    pltpu.sync_copy(acc.at[pl.ds(s * STRIPE, STRIPE), pl.ds(0, DEG_OUT)],
                    out_hbm.at[c].at[pl.ds(s * STRIPE, STRIPE)])


@functools.cache
def _deg_call():
    return pl.kernel(
        _deg_body,
        out_type=jax.ShapeDtypeStruct((NC, N_ACC, DEG_OUT), jnp.float32),
        mesh=plsc.VectorSubcoreMesh(core_axis_name="c", subcore_axis_name="s"),
        scratch_types=[
            pltpu.VMEM((CPT, CHUNK), jnp.int32),
            pltpu.VMEM((CHUNK, DW), jnp.float32),
            pltpu.VMEM_SHARED((N_ACC, DW), jnp.float32),
        ],
    )


def _prop_body(src2_hbm, dst1_hbm, y_hbm, out_hbm,
               srcall, dstv0, dstv1, rows0, rows1, acc,
               semg0, semg1, semd0, semd1):
    c = lax.axis_index("c")
    s = lax.axis_index("s")
    wid = c * NS + s
    base = wid * TE

    pltpu.sync_copy(src2_hbm.at[pl.ds(wid * CPT, CPT)], srcall)
    _fill(rows0, 0.0)

    @pl.loop(0, STRIPE // CHUNK)
    def _(z):
        pltpu.sync_copy(rows0, acc.at[pl.ds(s * STRIPE + z * CHUNK, CHUNK)])

    plsc.subcore_barrier()

    pltpu.async_copy(dst1_hbm.at[pl.ds(base, CHUNK)], dstv0, semd0)
    pltpu.async_copy(y_hbm.at[srcall.at[0]], rows0, semg0)
    pltpu.async_copy(dst1_hbm.at[pl.ds(base + CHUNK, CHUNK)], dstv1, semd1)

    @pl.loop(0, CPT, step=2)
    def _(k):
        pltpu.async_copy(y_hbm.at[srcall.at[k + 1]], rows1, semg1)

        pltpu.make_async_copy(dst1_hbm.at[pl.ds(base, CHUNK)], dstv0,
                              semd0).wait()
        pltpu.make_async_copy(y_hbm.at[srcall.at[k]], rows0, semg0).wait()
        pltpu.sync_copy(rows0, acc.at[dstv0], add=True)

        @pl.when(k + 2 < CPT)
        def _():
            pltpu.async_copy(dst1_hbm.at[pl.ds(base + (k + 2) * CHUNK, CHUNK)],
                             dstv0, semd0)
            pltpu.async_copy(y_hbm.at[srcall.at[k + 2]], rows0, semg0)

        pltpu.make_async_copy(dst1_hbm.at[pl.ds(base, CHUNK)], dstv1,
                              semd1).wait()
        pltpu.make_async_copy(y_hbm.at[srcall.at[k + 1]], rows1, semg1).wait()
        pltpu.sync_copy(rows1, acc.at[dstv1], add=True)

        @pl.when(k + 3 < CPT)
        def _():
            pltpu.async_copy(dst1_hbm.at[pl.ds(base + (k + 3) * CHUNK, CHUNK)],
                             dstv1, semd1)

    plsc.subcore_barrier()
    pltpu.sync_copy(acc.at[pl.ds(s * STRIPE, STRIPE)],
                    out_hbm.at[c].at[pl.ds(s * STRIPE, STRIPE)])


@functools.cache
def _prop_call():
    return pl.kernel(
        _prop_body,
        out_type=jax.ShapeDtypeStruct((NC, N_ACC, D), jnp.float32),
        mesh=plsc.VectorSubcoreMesh(core_axis_name="c", subcore_axis_name="s"),
        scratch_types=[
            pltpu.VMEM((CPT, CHUNK), jnp.int32),
            pltpu.VMEM((CHUNK,), jnp.int32),
            pltpu.VMEM((CHUNK,), jnp.int32),
            pltpu.VMEM((CHUNK, D), jnp.float32),
            pltpu.VMEM((CHUNK, D), jnp.float32),
            pltpu.VMEM_SHARED((N_ACC, D), jnp.float32),
            pltpu.SemaphoreType.DMA,
            pltpu.SemaphoreType.DMA,
            pltpu.SemaphoreType.DMA,
            pltpu.SemaphoreType.DMA,
        ],
    )


_BM = 1000
_NB = N // _BM
_NF = float(N)


def _prep_body(dp0, dp1, x_ref, y1_ref, dinv_ref):
    deg = dp0[0][:, 0:1] + dp1[0][:, 0:1]
    dinv = jnp.where(deg > 0.0, lax.rsqrt(jnp.where(deg > 0.0, deg, 1.0)), 0.0)
    dinv_ref[...] = dinv
    y1_ref[...] = x_ref[...] * dinv


def _mid_body(zp0, zp1, dinv_ref, h1_ref, y2_ref):
    dinv = dinv_ref[...]
    h1 = (zp0[0] + zp1[0]) * dinv
    h1_ref[...] = h1
    y2_ref[...] = h1 * dinv


def _dense_body(zp0, zp1, dinv_ref, x_ref, h1_ref,
                w0, b0, w1, b1, w2, b2, g1, bb1, l1w, l1b, l2w, l2b,
                g2, bb2, o_ref, out_scr, h_scr, st1, st2):
    p = pl.program_id(0)
    i = pl.program_id(1)
    dot = functools.partial(jnp.dot, preferred_element_type=jnp.float32,
                            precision=lax.Precision.DEFAULT)
    rows = pl.ds(i * _BM, _BM)

    @pl.when(p == 0)
    def _():
        @pl.when(i == 0)
        def _():
            st1[...] = jnp.zeros_like(st1)

        h2 = (zp0[0] + zp1[0]) * dinv_ref[...]
        o = jnp.concatenate([
            dot(x_ref[...], w0[...]) + b0[...],
            dot(h1_ref[...], w1[...]) + b1[...],
            dot(h2, w2[...]) + b2[...],
        ], axis=1)
        out_scr[rows, :] = o
        st1[0:1, :] += jnp.sum(o, axis=0, keepdims=True)
        st1[1:2, :] += jnp.sum(o * o, axis=0, keepdims=True)

    @pl.when(p == 1)
    def _():
        @pl.when(i == 0)
        def _():
            st2[...] = jnp.zeros_like(st2)

        mu = st1[0:1, :] / _NF
        var = st1[1:2, :] / _NF - mu * mu
        inv = lax.rsqrt(var + 1e-5)
        hb = (out_scr[rows, :] - mu) * inv * g1[...] + bb1[...]
        t = jnp.maximum(dot(hb, l1w[...]) + l1b[...], 0.0)
        y = dot(t, l2w[...]) + l2b[...]
        h = hb[:, :D] + y
        h_scr[rows, :] = h
        st2[0:1, :] += jnp.sum(h, axis=0, keepdims=True)
        st2[1:2, :] += jnp.sum(h * h, axis=0, keepdims=True)

    @pl.when(p == 2)
    def _():
        mu = st2[0:1, :] / _NF
        var = st2[1:2, :] / _NF - mu * mu
        inv = lax.rsqrt(var + 1e-5)
        o_ref[...] = (h_scr[rows, :] - mu) * inv * g2[...] + bb2[...]


def _row_spec(bn, bd):
    return pl.BlockSpec((bn, bd), lambda i: (i, 0))


def _part_spec(part, bn, bd):
    return pl.BlockSpec((1, bn, bd), lambda i, q=part: (q, i, 0))


def _full_spec(shape):
    return pl.BlockSpec(shape, lambda i: tuple(0 for _ in shape))


def _drow_spec(bn, bd):
    return pl.BlockSpec((bn, bd), lambda p, i: (jnp.where(p == 0, i, 0), 0))


def _dpart_spec(part, bn, bd):
    return pl.BlockSpec((1, bn, bd),
                        lambda p, i, q=part: (q, jnp.where(p == 0, i, 0), 0))


def _dfull_spec(shape):
    return pl.BlockSpec(shape, lambda p, i: tuple(0 for _ in shape))


def _dout_spec(bn, bd):
    return pl.BlockSpec((bn, bd), lambda p, i: (jnp.where(p == 2, i, 0), 0))


@jax.jit
def kernel(x, edge_index, W0, b0, W1, b1, W2, b2, bn1_g, bn1_b,
           l1_W, l1_b, l2_W, l2_b, bn2_g, bn2_b):
    E = edge_index.shape[1]
    pad_src = jnp.arange(E_PAD - E, dtype=jnp.int32) % N
    pad_dst = jnp.full((E_PAD - E,), PAD_ROW, jnp.int32)
    src = jnp.concatenate([edge_index[0].astype(jnp.int32), pad_src])
    dst1 = jnp.concatenate([edge_index[1].astype(jnp.int32), pad_dst])
    src2 = src.reshape(E_PAD // CHUNK, CHUNK)
    dst2 = dst1.reshape(E_PAD // CHUNK, CHUNK)


    b0r = b0.reshape(1, D)
    b1r = b1.reshape(1, D)
    b2r = b2.reshape(1, D)
    g1r = bn1_g.reshape(1, H3)
    bb1r = bn1_b.reshape(1, H3)
    l1br = l1_b.reshape(1, H)
    l2br = l2_b.reshape(1, D)
    g2r = bn2_g.reshape(1, D)
    bb2r = bn2_b.reshape(1, D)

    degp = _deg_call()(dst2)

    y1, dinv = pl.pallas_call(
        _prep_body,
        grid=(_NB,),
        in_specs=[_part_spec(0, _BM, DEG_OUT), _part_spec(1, _BM, DEG_OUT),
                  _row_spec(_BM, D)],
        out_specs=[_row_spec(_BM, D), _row_spec(_BM, 1)],
        out_shape=[jax.ShapeDtypeStruct((N_ACC, D), jnp.float32),
                   jax.ShapeDtypeStruct((N, 1), jnp.float32)],
    )(degp, degp, x)

    z1p = _prop_call()(src2, dst1, y1)

    h1, y2 = pl.pallas_call(
        _mid_body,
        grid=(_NB,),
        in_specs=[_part_spec(0, _BM, D), _part_spec(1, _BM, D),
                  _row_spec(_BM, 1)],
        out_specs=[_row_spec(_BM, D), _row_spec(_BM, D)],
        out_shape=[jax.ShapeDtypeStruct((N, D), jnp.float32),
                   jax.ShapeDtypeStruct((N_ACC, D), jnp.float32)],
    )(z1p, z1p, dinv)

    z2p = _prop_call()(src2, dst1, y2)

    final = pl.pallas_call(
        _dense_body,
        grid=(3, _NB),
        in_specs=[_dpart_spec(0, _BM, D), _dpart_spec(1, _BM, D),
                  _drow_spec(_BM, 1), _drow_spec(_BM, D), _drow_spec(_BM, D),
                  _dfull_spec((D, D)), _dfull_spec((1, D)),
                  _dfull_spec((D, D)), _dfull_spec((1, D)),
                  _dfull_spec((D, D)), _dfull_spec((1, D)),
                  _dfull_spec((1, H3)), _dfull_spec((1, H3)),
                  _dfull_spec((H3, H)), _dfull_spec((1, H)),
                  _dfull_spec((H, D)), _dfull_spec((1, D)),
                  _dfull_spec((1, D)), _dfull_spec((1, D))],
        out_specs=_dout_spec(_BM, D),
        out_shape=jax.ShapeDtypeStruct((N, D), jnp.float32),
        scratch_shapes=[pltpu.VMEM((N, H3), jnp.float32),
                        pltpu.VMEM((N, D), jnp.float32),
                        pltpu.VMEM((8, H3), jnp.float32),
                        pltpu.VMEM((8, D), jnp.float32)],
    )(z2p, z2p, dinv, x, h1, W0, b0r, W1, b1r, W2, b2r,
      g1r, bb1r, l1_W, l1br, l2_W, l2br, g2r, bb2r)

    return final
